# baseline (speedup 1.0000x reference)
import functools

import jax
import jax.numpy as jnp
from jax import lax
from jax.experimental import pallas as pl
from jax.experimental.pallas import tpu as pltpu
from jax.experimental.pallas import tpu_sc as plsc

NU = 10000
NM = 10000
D = 128
R = 5
E = 100000
N = R * E
K = 40
CH = N // K
NS = 16
TRIPS = (CH + NS - 1) // NS
ROWS_PER_TILE = 624


def _dense_body(rfeat_ref, pwu_ref, swu_ref, rwu_ref, pwm_ref, swm_ref, rwm_ref,
                rf0_ref, pa0_ref, rf1_ref, pa1_ref):
    x = rfeat_ref[0]
    for rw_ref, sw_ref, pw_ref, rf_ref, pa_ref in (
            (rwu_ref, swu_ref, pwu_ref, rf0_ref, pa0_ref),
            (rwm_ref, swm_ref, pwm_ref, rf1_ref, pa1_ref)):
        rw = rw_ref[0]
        rf = lax.dot_general(x, rw, (((1,), (1,)), ((), ())),
                             preferred_element_type=jnp.float32)
        sg = jax.nn.sigmoid(x @ sw_ref[0, 0])
        pa = jax.nn.sigmoid(x @ pw_ref[0, 0])
        rf_ref[0] = rf * sg[:, None]
        pa_ref[0] = jnp.broadcast_to(pa[:, None], pa.shape + (16,))


def _dense_phase(review_feat, prob_w_um, score_w_um, review_w_um,
                 prob_w_mu, score_w_mu, review_w_mu):
    be = 1000
    grid = (R, E // be)
    return pl.pallas_call(
        _dense_body,
        grid=grid,
        in_specs=[
            pl.BlockSpec((1, be, D), lambda r, b: (r, b, 0)),
            pl.BlockSpec((1, 1, D), lambda r, b: (r, 0, 0)),
            pl.BlockSpec((1, 1, D), lambda r, b: (r, 0, 0)),
            pl.BlockSpec((1, D, D), lambda r, b: (r, 0, 0)),
            pl.BlockSpec((1, 1, D), lambda r, b: (r, 0, 0)),
            pl.BlockSpec((1, 1, D), lambda r, b: (r, 0, 0)),
            pl.BlockSpec((1, D, D), lambda r, b: (r, 0, 0)),
        ],
        out_specs=[
            pl.BlockSpec((1, be, D), lambda r, b: (r, b, 0)),
            pl.BlockSpec((1, be, 16), lambda r, b: (r, b, 0)),
            pl.BlockSpec((1, be, D), lambda r, b: (r, b, 0)),
            pl.BlockSpec((1, be, 16), lambda r, b: (r, b, 0)),
        ],
        out_shape=[
            jax.ShapeDtypeStruct((R, E, D), jnp.float32),
            jax.ShapeDtypeStruct((R, E, 16), jnp.float32),
            jax.ShapeDtypeStruct((R, E, D), jnp.float32),
            jax.ShapeDtypeStruct((R, E, 16), jnp.float32),
        ],
    )(review_feat, prob_w_um[:, None, :], score_w_um[:, None, :], review_w_um,
      prob_w_mu[:, None, :], score_w_mu[:, None, :], review_w_mu)


def _wx_body(wu_ref, wm_ref, ucj_ref, mcj_ref, wx0_ref, wx1_ref):
    ucj = ucj_ref[...]
    mcj = mcj_ref[...]
    wx0_ref[0, :, :D] = wu_ref[0] * ucj
    wx0_ref[0, :, D:] = jnp.broadcast_to(ucj, (ucj.shape[0], D))
    wx1_ref[0, :, :D] = wm_ref[0] * mcj
    wx1_ref[0, :, D:] = jnp.broadcast_to(mcj, (mcj.shape[0], D))


def _wx_phase(W_user, W_movie, user_cj, movie_cj):
    bv = 1000
    return pl.pallas_call(
        _wx_body,
        grid=(R, NU // bv),
        in_specs=[
            pl.BlockSpec((1, bv, D), lambda r, b: (r, b, 0)),
            pl.BlockSpec((1, bv, D), lambda r, b: (r, b, 0)),
            pl.BlockSpec((bv, 1), lambda r, b: (b, 0)),
            pl.BlockSpec((bv, 1), lambda r, b: (b, 0)),
        ],
        out_specs=[
            pl.BlockSpec((1, bv, 2 * D), lambda r, b: (r, b, 0)),
            pl.BlockSpec((1, bv, 2 * D), lambda r, b: (r, b, 0)),
        ],
        out_shape=[
            jax.ShapeDtypeStruct((R, NU, 2 * D), jnp.float32),
            jax.ShapeDtypeStruct((R, NM, 2 * D), jnp.float32),
        ],
    )(W_user, W_movie, user_cj, movie_cj)


def _sc_body(Wx0, Wx1,
             widx0, sidx0, pa0, rf0,
             widx1, sidx1, pa1, rf1,
             ufeat_out, ifeat_out,
             widx_a, sidx_a, pa_a, rf_a, w_a,
             widx_b, sidx_b, pa_b, rf_b, w_b,
             zb_v, acc,
             lsem_a, lsem_b, gsem_a, gsem_b, ssem_a, ssem_b):
    bufs = ((widx_a, sidx_a, pa_a, rf_a, w_a, lsem_a, gsem_a, ssem_a),
            (widx_b, sidx_b, pa_b, rf_b, w_b, lsem_b, gsem_b, ssem_b))
    c = lax.axis_index("c")
    s = lax.axis_index("s")

    def _zrow(i, carry):
        for l in range(8):
            zb_v[i, pl.ds(l * 16, 16)] = jnp.zeros((16,), jnp.float32)
        return carry
    lax.fori_loop(0, 16, _zrow, 0)
    ntrips = jnp.where(s == NS - 1, 40, 39)

    def _zcopy(j, carry):
        pltpu.sync_copy(zb_v, acc.at[pl.ds(s * ROWS_PER_TILE + j * 16, 16)])
        return carry
    lax.fori_loop(0, ntrips, _zcopy, 0)
    plsc.subcore_barrier()

    def _process(widx_hbm, sidx_hbm, pa_hbm, rf_hbm, wtab_hbm):
        def _linear_descs(buf, ci):
            widx_v, sidx_v, pa_v, rf_v, w_v, lsem, gsem, ssem = buf
            base = ci * K
            return lsem, (
                (widx_hbm.at[pl.ds(base, K)], widx_v),
                (sidx_hbm.at[pl.ds(base, K)], sidx_v),
                (pa_hbm.at[pl.ds(base, K)], pa_v),
                (rf_hbm.at[pl.ds(base, K)], rf_v),
            )

        def _issue_linear(buf, ci):
            lsem, descs = _linear_descs(buf, ci)
            for src, dst in descs:
                pltpu.async_copy(src, dst, lsem)

        def _wait_linear(buf, ci):
            lsem, descs = _linear_descs(buf, ci)
            for src, dst in descs:
                pltpu.make_async_copy(src, dst, lsem).wait()

        def _gather_descs(buf):
            widx_v, sidx_v, pa_v, rf_v, w_v, lsem, gsem, ssem = buf
            return gsem, ((wtab_hbm.at[widx_v], w_v),)

        def _issue_gather(buf):
            gsem, descs = _gather_descs(buf)
            for src, dst in descs:
                pltpu.async_copy(src, dst, gsem)

        def _wait_gather(buf):
            gsem, descs = _gather_descs(buf)
            for src, dst in descs:
                pltpu.make_async_copy(src, dst, gsem).wait()

        def _scatter_desc(buf):
            widx_v, sidx_v, pa_v, rf_v, w_v, lsem, gsem, ssem = buf
            return (rf_v, acc.at[sidx_v], ssem)

        def _mask_sidx(sidx_v, valid):
            dump = NU + lax.iota(jnp.int32, 16)
            offs = list(range(0, K - 15, 16))
            if K % 16:
                offs.append(K - 16)
            for off in offs:
                sl = pl.ds(off, 16)
                sidx_v[sl] = jnp.where(valid, sidx_v[sl], dump)

        def _do_chunk(buf, ci):
            widx_v, sidx_v, pa_v, rf_v, w_v, lsem, gsem, ssem = buf
            nbuf = bufs[1] if buf is bufs[0] else bufs[0]
            nci_c = jnp.minimum(ci + NS, CH - 1)

            src, dst, ssem_n = _scatter_desc(nbuf)
            pltpu.make_async_copy(src, dst, ssem_n).wait()
            _issue_linear(nbuf, nci_c)

            _wait_gather(buf)

            def _edge(e, carry2):
                pa_s = pa_v[e, :]
                cj_s = w_v[e, pl.ds(D, 16)]
                for l in range(8):
                    sl = pl.ds(l * 16, 16)
                    rf_v[e, sl] = (w_v[e, sl] * pa_s
                                   + rf_v[e, sl] * cj_s)
                return carry2
            lax.fori_loop(0, K, _edge, 0)

            _wait_linear(nbuf, nci_c)
            _mask_sidx(nbuf[1], (ci + NS) < CH)
            _issue_gather(nbuf)

            src, dst, ssem_c = _scatter_desc(buf)
            pltpu.async_copy(src, dst, ssem_c, add=True)

        _issue_linear(bufs[0], s)
        _wait_linear(bufs[0], s)
        _mask_sidx(bufs[0][1], s < CH)
        _issue_gather(bufs[0])
        _mask_sidx(bufs[1][1], jnp.bool_(False))
        src0, dst0, ssem0 = _scatter_desc(bufs[1])
        pltpu.async_copy(src0, dst0, ssem0, add=True)

        def _pair_trip(m, carry):
            _do_chunk(bufs[0], s + (2 * m) * NS)
            _do_chunk(bufs[1], s + (2 * m + 1) * NS)
            return carry
        lax.fori_loop(0, (TRIPS + 1) // 2, _pair_trip, 0)
        _wait_gather(bufs[0])
        srcl, dstl, sseml = _scatter_desc(bufs[1])
        pltpu.make_async_copy(srcl, dstl, sseml).wait()

    @pl.when(c == 0)
    def _():
        _process(widx0, sidx0, pa0, rf0, Wx0)

    @pl.when(c == 1)
    def _():
        _process(widx1, sidx1, pa1, rf1, Wx1)

    plsc.subcore_barrier()

    @pl.when(c == 0)
    def _():
        def _fcopy(j, carry):
            off = s * ROWS_PER_TILE + j * 16
            pltpu.sync_copy(acc.at[pl.ds(off, 16)],
                            ifeat_out.at[pl.ds(off, 16)])
            return carry
        lax.fori_loop(0, ntrips, _fcopy, 0)

    @pl.when(c == 1)
    def _():
        def _fcopy(j, carry):
            off = s * ROWS_PER_TILE + j * 16
            pltpu.sync_copy(acc.at[pl.ds(off, 16)],
                            ufeat_out.at[pl.ds(off, 16)])
            return carry
        lax.fori_loop(0, ntrips, _fcopy, 0)


def _sparse_phase(Wx0, Wx1, d0, d1):
    mesh = plsc.VectorSubcoreMesh(core_axis_name="c", subcore_axis_name="s")
    fn = pl.kernel(
        _sc_body,
        out_type=(jax.ShapeDtypeStruct((NU, D), jnp.float32),
                  jax.ShapeDtypeStruct((NM, D), jnp.float32)),
        mesh=mesh,
        scratch_types=(
            [pltpu.VMEM((K,), jnp.int32)] * 2
            + [pltpu.VMEM((K, 16), jnp.float32)]
            + [pltpu.VMEM((K, D), jnp.float32)]
            + [pltpu.VMEM((K, 2 * D), jnp.float32)]
        ) * 2 + [
            pltpu.VMEM((16, D), jnp.float32),
            pltpu.VMEM_SHARED((NU + 16, D), jnp.float32),
        ] + [pltpu.SemaphoreType.DMA] * 6,
    )
    return fn(Wx0, Wx1, *d0, *d1)


def _gelu_exact(x):
    return x * 0.5 * (1.0 + lax.erf(x * 0.7071067811865476))


def _tail_body(uf_ref, if_ref, uci_ref, ici_ref, uW_ref, ub_ref, iW_ref, ib_ref,
               uo_ref, io_ref):
    uf = _gelu_exact(uf_ref[...] * uci_ref[...])
    io = _gelu_exact(if_ref[...] * ici_ref[...])
    uo_ref[...] = uf @ uW_ref[...].T + ub_ref[...][None, :]
    io_ref[...] = io @ iW_ref[...].T + ib_ref[...][None, :]


def _tail_phase(ufeat, ifeat, user_ci, movie_ci, ufc_W, ufc_b, ifc_W, ifc_b):
    grid = 10
    blk_u = NU // grid
    blk_m = NM // grid
    return pl.pallas_call(
        _tail_body,
        grid=(grid,),
        in_specs=[
            pl.BlockSpec((blk_u, D), lambda i: (i, 0)),
            pl.BlockSpec((blk_m, D), lambda i: (i, 0)),
            pl.BlockSpec((blk_u, 1), lambda i: (i, 0)),
            pl.BlockSpec((blk_m, 1), lambda i: (i, 0)),
            pl.BlockSpec((D, D), lambda i: (0, 0)),
            pl.BlockSpec((D,), lambda i: (0,)),
            pl.BlockSpec((D, D), lambda i: (0, 0)),
            pl.BlockSpec((D,), lambda i: (0,)),
        ],
        out_specs=[
            pl.BlockSpec((blk_u, D), lambda i: (i, 0)),
            pl.BlockSpec((blk_m, D), lambda i: (i, 0)),
        ],
        out_shape=[
            jax.ShapeDtypeStruct((NU, D), jnp.float32),
            jax.ShapeDtypeStruct((NM, D), jnp.float32),
        ],
    )(ufeat, ifeat, user_ci, movie_ci, ufc_W, ufc_b, ifc_W, ifc_b)


def kernel(edge_index, review_feat, user_cj, user_ci, movie_cj, movie_ci,
           W_user, W_movie, prob_w_um, score_w_um, review_w_um,
           prob_w_mu, score_w_mu, review_w_mu, ufc_W, ufc_b, ifc_W, ifc_b):
    rf0, pa0, rf1, pa1 = _dense_phase(
        review_feat, prob_w_um, score_w_um, review_w_um,
        prob_w_mu, score_w_mu, review_w_mu)

    Wx0, Wx1 = _wx_phase(W_user, W_movie, user_cj, movie_cj)

    src = edge_index[:, 0, :].astype(jnp.int32)
    dst = edge_index[:, 1, :].astype(jnp.int32)
    roffs = (jnp.arange(R, dtype=jnp.int32) * NU)[:, None]
    d0 = ((src + roffs).reshape(N), dst.reshape(N),
          pa0.reshape(N, 16), rf0.reshape(N, D))
    d1 = ((dst + roffs).reshape(N), src.reshape(N),
          pa1.reshape(N, 16), rf1.reshape(N, D))

    ufeat, ifeat = _sparse_phase(
        Wx0.reshape(R * NU, 2 * D), Wx1.reshape(R * NM, 2 * D), d0, d1)

    return _tail_phase(ufeat, ifeat, user_ci, movie_ci,
                       ufc_W, ufc_b, ifc_W, ifc_b)


# speedup vs baseline: 1.0082x; 1.0082x over previous
import functools

import jax
import jax.numpy as jnp
from jax import lax
from jax.experimental import pallas as pl
from jax.experimental.pallas import tpu as pltpu
from jax.experimental.pallas import tpu_sc as plsc

NU = 10000
NM = 10000
D = 128
R = 5
E = 100000
N = R * E
K = 40
CH = N // K
NS = 16
TRIPS = (CH + NS - 1) // NS
ROWS_PER_TILE = 624


def _dense_body(rfeat_ref, pwu_ref, swu_ref, rwu_ref, pwm_ref, swm_ref, rwm_ref,
                rf0_ref, pa0_ref, rf1_ref, pa1_ref):
    x = rfeat_ref[0]
    for rw_ref, sw_ref, pw_ref, rf_ref, pa_ref in (
            (rwu_ref, swu_ref, pwu_ref, rf0_ref, pa0_ref),
            (rwm_ref, swm_ref, pwm_ref, rf1_ref, pa1_ref)):
        rw = rw_ref[0]
        rf = lax.dot_general(x, rw, (((1,), (1,)), ((), ())),
                             preferred_element_type=jnp.float32)
        sg = jax.nn.sigmoid(x @ sw_ref[0, 0])
        pa = jax.nn.sigmoid(x @ pw_ref[0, 0])
        rf_ref[0] = rf * sg[:, None]
        pa_ref[0] = jnp.broadcast_to(pa[:, None], pa.shape + (16,))


def _dense_phase(review_feat, prob_w_um, score_w_um, review_w_um,
                 prob_w_mu, score_w_mu, review_w_mu):
    be = 1000
    grid = (R, E // be)
    return pl.pallas_call(
        _dense_body,
        grid=grid,
        in_specs=[
            pl.BlockSpec((1, be, D), lambda r, b: (r, b, 0)),
            pl.BlockSpec((1, 1, D), lambda r, b: (r, 0, 0)),
            pl.BlockSpec((1, 1, D), lambda r, b: (r, 0, 0)),
            pl.BlockSpec((1, D, D), lambda r, b: (r, 0, 0)),
            pl.BlockSpec((1, 1, D), lambda r, b: (r, 0, 0)),
            pl.BlockSpec((1, 1, D), lambda r, b: (r, 0, 0)),
            pl.BlockSpec((1, D, D), lambda r, b: (r, 0, 0)),
        ],
        out_specs=[
            pl.BlockSpec((1, be, D), lambda r, b: (r, b, 0)),
            pl.BlockSpec((1, be, 16), lambda r, b: (r, b, 0)),
            pl.BlockSpec((1, be, D), lambda r, b: (r, b, 0)),
            pl.BlockSpec((1, be, 16), lambda r, b: (r, b, 0)),
        ],
        out_shape=[
            jax.ShapeDtypeStruct((R, E, D), jnp.float32),
            jax.ShapeDtypeStruct((R, E, 16), jnp.float32),
            jax.ShapeDtypeStruct((R, E, D), jnp.float32),
            jax.ShapeDtypeStruct((R, E, 16), jnp.float32),
        ],
    )(review_feat, prob_w_um[:, None, :], score_w_um[:, None, :], review_w_um,
      prob_w_mu[:, None, :], score_w_mu[:, None, :], review_w_mu)


def _wx_body(wu_ref, wm_ref, ucj_ref, mcj_ref, wx0_ref, wx1_ref):
    ucj = ucj_ref[...]
    mcj = mcj_ref[...]
    wx0_ref[0, :, :D] = wu_ref[0] * ucj
    wx0_ref[0, :, D:] = jnp.broadcast_to(ucj, (ucj.shape[0], D))
    wx1_ref[0, :, :D] = wm_ref[0] * mcj
    wx1_ref[0, :, D:] = jnp.broadcast_to(mcj, (mcj.shape[0], D))


def _wx_phase(W_user, W_movie, user_cj, movie_cj):
    bv = 1000
    return pl.pallas_call(
        _wx_body,
        grid=(R, NU // bv),
        in_specs=[
            pl.BlockSpec((1, bv, D), lambda r, b: (r, b, 0)),
            pl.BlockSpec((1, bv, D), lambda r, b: (r, b, 0)),
            pl.BlockSpec((bv, 1), lambda r, b: (b, 0)),
            pl.BlockSpec((bv, 1), lambda r, b: (b, 0)),
        ],
        out_specs=[
            pl.BlockSpec((1, bv, 2 * D), lambda r, b: (r, b, 0)),
            pl.BlockSpec((1, bv, 2 * D), lambda r, b: (r, b, 0)),
        ],
        out_shape=[
            jax.ShapeDtypeStruct((R, NU, 2 * D), jnp.float32),
            jax.ShapeDtypeStruct((R, NM, 2 * D), jnp.float32),
        ],
    )(W_user, W_movie, user_cj, movie_cj)


def _sc_body(Wx0, Wx1,
             widx0, sidx0, pa0, rf0,
             widx1, sidx1, pa1, rf1,
             ufeat_out, ifeat_out,
             widx_a, sidx_a, pa_a, rf_a, w_a,
             widx_b, sidx_b, pa_b, rf_b, w_b,
             zb_v, acc,
             lsem_a, lsem_b, gsem_a, gsem_b, ssem_a, ssem_b):
    bufs = ((widx_a, sidx_a, pa_a, rf_a, w_a, lsem_a, gsem_a, ssem_a),
            (widx_b, sidx_b, pa_b, rf_b, w_b, lsem_b, gsem_b, ssem_b))
    c = lax.axis_index("c")
    s = lax.axis_index("s")

    def _zrow(i, carry):
        for l in range(8):
            zb_v[i, pl.ds(l * 16, 16)] = jnp.zeros((16,), jnp.float32)
        return carry
    lax.fori_loop(0, 16, _zrow, 0)
    ntrips = jnp.where(s == NS - 1, 40, 39)

    def _zcopy(j, carry):
        pltpu.sync_copy(zb_v, acc.at[pl.ds(s * ROWS_PER_TILE + j * 16, 16)])
        return carry
    lax.fori_loop(0, ntrips, _zcopy, 0)
    plsc.subcore_barrier()

    def _process(widx_hbm, sidx_hbm, pa_hbm, rf_hbm, wtab_hbm):
        def _linear_descs(buf, ci):
            widx_v, sidx_v, pa_v, rf_v, w_v, lsem, gsem, ssem = buf
            base = ci * K
            return lsem, (
                (widx_hbm.at[pl.ds(base, K)], widx_v),
                (sidx_hbm.at[pl.ds(base, K)], sidx_v),
                (pa_hbm.at[pl.ds(base, K)], pa_v),
                (rf_hbm.at[pl.ds(base, K)], rf_v),
            )

        def _issue_linear(buf, ci):
            lsem, descs = _linear_descs(buf, ci)
            for src, dst in descs:
                pltpu.async_copy(src, dst, lsem)

        def _wait_linear(buf, ci):
            lsem, descs = _linear_descs(buf, ci)
            for src, dst in descs:
                pltpu.make_async_copy(src, dst, lsem).wait()

        def _gather_descs(buf):
            widx_v, sidx_v, pa_v, rf_v, w_v, lsem, gsem, ssem = buf
            return gsem, ((wtab_hbm.at[widx_v], w_v),)

        def _issue_gather(buf):
            gsem, descs = _gather_descs(buf)
            for src, dst in descs:
                pltpu.async_copy(src, dst, gsem)

        def _wait_gather(buf):
            gsem, descs = _gather_descs(buf)
            for src, dst in descs:
                pltpu.make_async_copy(src, dst, gsem).wait()

        def _scatter_desc(buf):
            widx_v, sidx_v, pa_v, rf_v, w_v, lsem, gsem, ssem = buf
            return (rf_v, acc.at[sidx_v], ssem)

        def _mask_sidx(sidx_v, valid):
            dump = NU + lax.iota(jnp.int32, 16)
            offs = list(range(0, K - 15, 16))
            if K % 16:
                offs.append(K - 16)
            for off in offs:
                sl = pl.ds(off, 16)
                sidx_v[sl] = jnp.where(valid, sidx_v[sl], dump)

        def _do_chunk(buf, ci):
            widx_v, sidx_v, pa_v, rf_v, w_v, lsem, gsem, ssem = buf
            nbuf = bufs[1] if buf is bufs[0] else bufs[0]
            nci_c = jnp.minimum(ci + NS, CH - 1)

            src, dst, ssem_n = _scatter_desc(nbuf)
            pltpu.make_async_copy(src, dst, ssem_n).wait()
            _issue_linear(nbuf, nci_c)

            _wait_gather(buf)

            def _eblock(eb, carry2):
                e0 = eb * 8
                for de in range(8):
                    e = e0 + de
                    pa_s = pa_v[e, :]
                    cj_s = w_v[e, pl.ds(D, 16)]
                    for l in range(8):
                        sl = pl.ds(l * 16, 16)
                        rf_v[e, sl] = (w_v[e, sl] * pa_s
                                       + rf_v[e, sl] * cj_s)
                return carry2
            lax.fori_loop(0, K // 8, _eblock, 0)

            _wait_linear(nbuf, nci_c)
            _mask_sidx(nbuf[1], (ci + NS) < CH)
            _issue_gather(nbuf)

            src, dst, ssem_c = _scatter_desc(buf)
            pltpu.async_copy(src, dst, ssem_c, add=True)

        _issue_linear(bufs[0], s)
        _wait_linear(bufs[0], s)
        _mask_sidx(bufs[0][1], s < CH)
        _issue_gather(bufs[0])
        _mask_sidx(bufs[1][1], jnp.bool_(False))
        src0, dst0, ssem0 = _scatter_desc(bufs[1])
        pltpu.async_copy(src0, dst0, ssem0, add=True)

        def _pair_trip(m, carry):
            _do_chunk(bufs[0], s + (2 * m) * NS)
            _do_chunk(bufs[1], s + (2 * m + 1) * NS)
            return carry
        lax.fori_loop(0, (TRIPS + 1) // 2, _pair_trip, 0)
        _wait_gather(bufs[0])
        srcl, dstl, sseml = _scatter_desc(bufs[1])
        pltpu.make_async_copy(srcl, dstl, sseml).wait()

    @pl.when(c == 0)
    def _():
        _process(widx0, sidx0, pa0, rf0, Wx0)

    @pl.when(c == 1)
    def _():
        _process(widx1, sidx1, pa1, rf1, Wx1)

    plsc.subcore_barrier()

    @pl.when(c == 0)
    def _():
        def _fcopy(j, carry):
            off = s * ROWS_PER_TILE + j * 16
            pltpu.sync_copy(acc.at[pl.ds(off, 16)],
                            ifeat_out.at[pl.ds(off, 16)])
            return carry
        lax.fori_loop(0, ntrips, _fcopy, 0)

    @pl.when(c == 1)
    def _():
        def _fcopy(j, carry):
            off = s * ROWS_PER_TILE + j * 16
            pltpu.sync_copy(acc.at[pl.ds(off, 16)],
                            ufeat_out.at[pl.ds(off, 16)])
            return carry
        lax.fori_loop(0, ntrips, _fcopy, 0)


def _sparse_phase(Wx0, Wx1, d0, d1):
    mesh = plsc.VectorSubcoreMesh(core_axis_name="c", subcore_axis_name="s")
    fn = pl.kernel(
        _sc_body,
        out_type=(jax.ShapeDtypeStruct((NU, D), jnp.float32),
                  jax.ShapeDtypeStruct((NM, D), jnp.float32)),
        mesh=mesh,
        scratch_types=(
            [pltpu.VMEM((K,), jnp.int32)] * 2
            + [pltpu.VMEM((K, 16), jnp.float32)]
            + [pltpu.VMEM((K, D), jnp.float32)]
            + [pltpu.VMEM((K, 2 * D), jnp.float32)]
        ) * 2 + [
            pltpu.VMEM((16, D), jnp.float32),
            pltpu.VMEM_SHARED((NU + 16, D), jnp.float32),
        ] + [pltpu.SemaphoreType.DMA] * 6,
    )
    return fn(Wx0, Wx1, *d0, *d1)


def _gelu_exact(x):
    return x * 0.5 * (1.0 + lax.erf(x * 0.7071067811865476))


def _tail_body(uf_ref, if_ref, uci_ref, ici_ref, uW_ref, ub_ref, iW_ref, ib_ref,
               uo_ref, io_ref):
    uf = _gelu_exact(uf_ref[...] * uci_ref[...])
    io = _gelu_exact(if_ref[...] * ici_ref[...])
    uo_ref[...] = uf @ uW_ref[...].T + ub_ref[...][None, :]
    io_ref[...] = io @ iW_ref[...].T + ib_ref[...][None, :]


def _tail_phase(ufeat, ifeat, user_ci, movie_ci, ufc_W, ufc_b, ifc_W, ifc_b):
    grid = 10
    blk_u = NU // grid
    blk_m = NM // grid
    return pl.pallas_call(
        _tail_body,
        grid=(grid,),
        in_specs=[
            pl.BlockSpec((blk_u, D), lambda i: (i, 0)),
            pl.BlockSpec((blk_m, D), lambda i: (i, 0)),
            pl.BlockSpec((blk_u, 1), lambda i: (i, 0)),
            pl.BlockSpec((blk_m, 1), lambda i: (i, 0)),
            pl.BlockSpec((D, D), lambda i: (0, 0)),
            pl.BlockSpec((D,), lambda i: (0,)),
            pl.BlockSpec((D, D), lambda i: (0, 0)),
            pl.BlockSpec((D,), lambda i: (0,)),
        ],
        out_specs=[
            pl.BlockSpec((blk_u, D), lambda i: (i, 0)),
            pl.BlockSpec((blk_m, D), lambda i: (i, 0)),
        ],
        out_shape=[
            jax.ShapeDtypeStruct((NU, D), jnp.float32),
            jax.ShapeDtypeStruct((NM, D), jnp.float32),
        ],
    )(ufeat, ifeat, user_ci, movie_ci, ufc_W, ufc_b, ifc_W, ifc_b)


def kernel(edge_index, review_feat, user_cj, user_ci, movie_cj, movie_ci,
           W_user, W_movie, prob_w_um, score_w_um, review_w_um,
           prob_w_mu, score_w_mu, review_w_mu, ufc_W, ufc_b, ifc_W, ifc_b):
    rf0, pa0, rf1, pa1 = _dense_phase(
        review_feat, prob_w_um, score_w_um, review_w_um,
        prob_w_mu, score_w_mu, review_w_mu)

    Wx0, Wx1 = _wx_phase(W_user, W_movie, user_cj, movie_cj)

    src = edge_index[:, 0, :].astype(jnp.int32)
    dst = edge_index[:, 1, :].astype(jnp.int32)
    roffs = (jnp.arange(R, dtype=jnp.int32) * NU)[:, None]
    d0 = ((src + roffs).reshape(N), dst.reshape(N),
          pa0.reshape(N, 16), rf0.reshape(N, D))
    d1 = ((dst + roffs).reshape(N), src.reshape(N),
          pa1.reshape(N, 16), rf1.reshape(N, D))

    ufeat, ifeat = _sparse_phase(
        Wx0.reshape(R * NU, 2 * D), Wx1.reshape(R * NM, 2 * D), d0, d1)

    return _tail_phase(ufeat, ifeat, user_ci, movie_ci,
                       ufc_W, ufc_b, ifc_W, ifc_b)
